# Initial kernel scaffold; baseline (speedup 1.0000x reference)
import functools

import jax
import jax.numpy as jnp
from jax import lax
from jax.experimental import pallas as pl
from jax.experimental.pallas import tpu as pltpu
from jax.experimental.pallas import tpu_sc as plsc

N = 10000
E = 320000
D = 128
R = 4

NPAD = 10240
NPASS = 4
RANGE = NPAD // NPASS
SLOTS = R * RANGE
TRASH = SLOTS
ACC_ROWS = 12288
CNT_ROWS = 10496
NC = 2
NS = 16
NT = NC * NS
EPT = E // NT
NV = EPT // 16
C = 128
SELCAP = 10240


def _sc_body(with_cnt, h, srcs, dsts, ets, z128, z16, o16, *rest):
    if with_cnt:
        a_out, cnt_out = rest[0], rest[1]
        scr = rest[2:]
    else:
        a_out = rest[0]
        scr = rest[1:]
    (e_src, e_dst, e_et, src_sel, slot_sel, src_chunk, slot_chunk, rows,
     zb, zs16, ones, acc, cacc, sem) = scr

    ci = lax.axis_index("c")
    si = lax.axis_index("s")
    tid = ci * NS + si
    base = tid * EPT

    pltpu.sync_copy(srcs.at[pl.ds(base, EPT)], e_src)
    pltpu.sync_copy(dsts.at[pl.ds(base, EPT)], e_dst)
    pltpu.sync_copy(ets.at[pl.ds(base, EPT)], e_et)
    pltpu.sync_copy(z128, zb)
    if with_cnt:
        pltpu.sync_copy(z16, zs16)
        pltpu.sync_copy(o16, ones)

    for rho in range(NPASS):
        for k in range(ACC_ROWS // NS // 128):
            pltpu.sync_copy(zb, acc.at[pl.ds(si * (ACC_ROWS // NS) + k * 128, 128)])
        if with_cnt:
            pltpu.sync_copy(zs16, cacc.at[pl.ds(si * (CNT_ROWS // NS), CNT_ROWS // NS)])
        plsc.subcore_barrier()

        lo = rho * RANGE

        def scan_body(i, cnt):
            dst_v = e_dst[pl.ds(i * 16, 16)]
            et_v = e_et[pl.ds(i * 16, 16)]
            src_v = e_src[pl.ds(i * 16, 16)]
            m = (dst_v >= lo) & (dst_v < lo + RANGE)
            slot_v = et_v * RANGE + (dst_v - lo)
            plsc.store_compressed(src_sel.at[pl.ds(cnt, 16)], src_v, mask=m)
            plsc.store_compressed(slot_sel.at[pl.ds(cnt, 16)], slot_v, mask=m)
            return cnt + jnp.sum(m.astype(jnp.int32))

        cnt = lax.fori_loop(0, NV, scan_body, jnp.int32(0))

        zero_vi = jnp.zeros((16,), jnp.int32)
        trash_v = jnp.full((16,), TRASH, jnp.int32)
        for k in range(C // 16):
            src_sel[pl.ds(cnt + k * 16, 16)] = zero_vi
            slot_sel[pl.ds(cnt + k * 16, 16)] = trash_v

        nch = (cnt + (C - 1)) // C

        def chunk_body(j, carry):
            pltpu.sync_copy(src_sel.at[pl.ds(j * C, C)], src_chunk)
            pltpu.sync_copy(slot_sel.at[pl.ds(j * C, C)], slot_chunk)
            pltpu.async_copy(h.at[src_chunk], rows, sem).wait()
            pltpu.sync_copy(rows, acc.at[slot_chunk], add=True)
            if with_cnt:
                pltpu.sync_copy(ones, cacc.at[slot_chunk], add=True)
            return carry

        lax.fori_loop(0, nch, chunk_body, jnp.int32(0))
        plsc.subcore_barrier()

        r_idx = si // 4
        noff = (si % 4) * 640
        pltpu.sync_copy(acc.at[pl.ds(si * 640, 640)],
                        a_out.at[ci, r_idx, pl.ds(lo + noff, 640)])
        if with_cnt:
            pltpu.sync_copy(cacc.at[pl.ds(si * 640, 640)],
                            cnt_out.at[ci, rho, pl.ds(si * 640, 640)])
        plsc.subcore_barrier()


def _make_sc(with_cnt):
    mesh = plsc.VectorSubcoreMesh(core_axis_name="c", subcore_axis_name="s")
    out_type = [jax.ShapeDtypeStruct((NC, R, NPAD, D), jnp.float32)]
    if with_cnt:
        out_type.append(jax.ShapeDtypeStruct((NC, NPASS, SLOTS, 16), jnp.float32))
    scratch = [
        pltpu.VMEM((EPT,), jnp.int32),
        pltpu.VMEM((EPT,), jnp.int32),
        pltpu.VMEM((EPT,), jnp.int32),
        pltpu.VMEM((SELCAP,), jnp.int32),
        pltpu.VMEM((SELCAP,), jnp.int32),
        pltpu.VMEM((C,), jnp.int32),
        pltpu.VMEM((C,), jnp.int32),
        pltpu.VMEM((C, D), jnp.float32),
        pltpu.VMEM((128, D), jnp.float32),
        pltpu.VMEM((CNT_ROWS // NS, 16), jnp.float32),
        pltpu.VMEM((C, 16), jnp.float32),
        pltpu.VMEM_SHARED((ACC_ROWS, D), jnp.float32),
        pltpu.VMEM_SHARED((CNT_ROWS, 16), jnp.float32),
        pltpu.SemaphoreType.DMA,
    ]
    return pl.kernel(
        functools.partial(_sc_body, with_cnt),
        out_type=tuple(out_type),
        mesh=mesh,
        scratch_types=scratch,
    )


def _tc_body(h_ref, a_ref, cnt_ref, wrel_ref, wroot_ref, bias_ref, g_ref,
             be_ref, o_ref):
    b = pl.program_id(0)
    h = h_ref[...]
    out = jnp.dot(h, wroot_ref[...], preferred_element_type=jnp.float32)
    out = out + bias_ref[...]
    for r in range(R):
        a = a_ref[0, r] + a_ref[1, r]
        c = cnt_ref[0, r, pl.ds(b, 1), :] + cnt_ref[1, r, pl.ds(b, 1), :]
        inv = (1.0 / jnp.maximum(c, 1.0)).reshape(D, 1)
        out = out + jnp.dot(a * inv, wrel_ref[r],
                            preferred_element_type=jnp.float32)
    mu = jnp.mean(out, axis=1, keepdims=True)
    xc = out - mu
    var = jnp.mean(xc * xc, axis=1, keepdims=True)
    y = xc * lax.rsqrt(var + 1e-5)
    y = y * g_ref[...] + be_ref[...]
    o_ref[...] = jnp.maximum(y, 0.0)


def _tc_layer(h, a_p, cnt_n, wrel, wroot, bias, g, be):
    nb = NPAD // 128
    return pl.pallas_call(
        _tc_body,
        grid=(nb,),
        in_specs=[
            pl.BlockSpec((128, D), lambda b: (b, 0)),
            pl.BlockSpec((NC, R, 128, D), lambda b: (0, 0, b, 0)),
            pl.BlockSpec((NC, R, nb, 128), lambda b: (0, 0, 0, 0)),
            pl.BlockSpec((R, D, D), lambda b: (0, 0, 0)),
            pl.BlockSpec((D, D), lambda b: (0, 0)),
            pl.BlockSpec((1, D), lambda b: (0, 0)),
            pl.BlockSpec((1, D), lambda b: (0, 0)),
            pl.BlockSpec((1, D), lambda b: (0, 0)),
        ],
        out_specs=pl.BlockSpec((128, D), lambda b: (b, 0)),
        out_shape=jax.ShapeDtypeStruct((NPAD, D), jnp.float32),
    )(h, a_p, cnt_n, wrel, wroot, bias, g, be)


_sc_l0 = _make_sc(True)
_sc_l1 = _make_sc(False)


def kernel(x, edge_index, edge_attr, Wrel0, Wroot0, b0, g0, be0,
           Wrel1, Wroot1, b1, g1, be1):
    src = edge_index[0]
    dst = edge_index[1]
    et = edge_attr[:, 0]

    h0 = jnp.pad(x, ((0, NPAD - N), (0, 0)))
    z128 = jnp.zeros((128, D), jnp.float32)
    z16 = jnp.zeros((CNT_ROWS // NS, 16), jnp.float32)
    o16 = jnp.ones((C, 16), jnp.float32)

    a0, cnt_raw = _sc_l0(h0, src, dst, et, z128, z16, o16)
    cnt = cnt_raw[..., 0].reshape(NC, NPASS, R, RANGE)
    cnt_n = jnp.transpose(cnt, (0, 2, 1, 3)).reshape(NC, R, NPAD // 128, 128)

    h1 = _tc_layer(h0, a0, cnt_n, Wrel0, Wroot0, b0.reshape(1, D),
                   g0.reshape(1, D), be0.reshape(1, D))
    (a1,) = _sc_l1(h1, src, dst, et, z128, z16, o16)
    h2 = _tc_layer(h1, a1, cnt_n, Wrel1, Wroot1, b1.reshape(1, D),
                   g1.reshape(1, D), be1.reshape(1, D))
    return jnp.stack([h1[:N], h2[:N]])


# speedup vs baseline: 5.8391x; 5.8391x over previous
import functools

import jax
import jax.numpy as jnp
from jax import lax
from jax.experimental import pallas as pl
from jax.experimental.pallas import tpu as pltpu
from jax.experimental.pallas import tpu_sc as plsc

N = 10000
E = 320000
D = 128
R = 4

NPAD = 10240
NPASS = 4
RANGE = NPAD // NPASS
SLOTS = R * RANGE
TRASH = SLOTS
ACC_ROWS = 10496
NC = 2
NS = 16
NT = NC * NS
EPT = E // NT
NV = EPT // 16
C = 128
NCH = 80
MASK14 = (1 << 14) - 1


def _sc_body(with_cnt, h, epk, z656, o128, *rest):
    if with_cnt:
        a_out, cnt_out = rest[0], rest[1]
        scr = rest[2:]
    else:
        a_out = rest[0]
        scr = rest[1:]
    (e_all, sel, src_chunk, slot_chunk, rows, acc, sem) = scr

    ci = lax.axis_index("c")
    si = lax.axis_index("s")
    tid = ci * NS + si

    pltpu.sync_copy(epk.at[pl.ds(tid * EPT, EPT)], e_all)

    nep = 2 * NPASS if with_cnt else NPASS
    for epoch in range(nep):
        rho = epoch % NPASS
        is_cnt = epoch >= NPASS

        pltpu.sync_copy(z656, acc.at[pl.ds(si * 656, 656)])
        if is_cnt and epoch == NPASS:
            pltpu.sync_copy(o128, rows)
        plsc.subcore_barrier()

        lo = rho * RANGE

        def scan_body(i, cnt):
            p = e_all[pl.ds(i * 16, 16)]
            src_v = jnp.bitwise_and(p, MASK14)
            dst_v = jnp.bitwise_and(jnp.right_shift(p, 14), MASK14)
            et_v = jnp.right_shift(p, 28)
            m = (dst_v >= lo) & (dst_v < lo + RANGE)
            slot_v = et_v * RANGE + (dst_v - lo)
            entry = jnp.bitwise_or(src_v, jnp.left_shift(slot_v, 14))
            mi = m.astype(jnp.int32)
            pf = plsc.cumsum(mi)
            q = cnt + pf - 1
            ri = jnp.right_shift(q, 7)
            co = jnp.bitwise_and(q, C - 1)
            plsc.store_scatter(sel, [ri, co], entry, mask=m)
            return cnt + jnp.sum(mi)

        cnt = lax.fori_loop(0, NV, scan_body, jnp.int32(0))

        it16 = lax.iota(jnp.int32, 16)
        trash_v = jnp.full((16,), TRASH << 14, jnp.int32)
        for k in range(C // 16):
            q = cnt + k * 16 + it16
            ri = jnp.right_shift(q, 7)
            co = jnp.bitwise_and(q, C - 1)
            plsc.store_scatter(sel, [ri, co], trash_v)

        nch = (cnt + (C - 1)) // C

        def chunk_body(j, carry):
            for k in range(C // 16):
                pp = sel[j, pl.ds(k * 16, 16)]
                src_chunk[pl.ds(k * 16, 16)] = jnp.bitwise_and(pp, MASK14)
                slot_chunk[pl.ds(k * 16, 16)] = jnp.right_shift(pp, 14)
            if not is_cnt:
                pltpu.async_copy(h.at[src_chunk], rows, sem).wait()
            pltpu.sync_copy(rows, acc.at[slot_chunk], add=True)
            return carry

        lax.fori_loop(0, nch, chunk_body, jnp.int32(0))
        plsc.subcore_barrier()

        r_idx = si // 4
        noff = (si % 4) * 640
        dst_ref = cnt_out if is_cnt else a_out
        pltpu.sync_copy(acc.at[pl.ds(si * 640, 640)],
                        dst_ref.at[ci, r_idx, pl.ds(lo + noff, 640)])
        plsc.subcore_barrier()


def _make_sc(with_cnt):
    mesh = plsc.VectorSubcoreMesh(core_axis_name="c", subcore_axis_name="s",
                                  num_cores=NC, num_subcores=NS)
    out_type = [jax.ShapeDtypeStruct((NC, R, NPAD, D), jnp.float32)]
    if with_cnt:
        out_type.append(
            jax.ShapeDtypeStruct((NC, R, NPAD, D), jnp.float32))
    scratch = [
        pltpu.VMEM((EPT,), jnp.int32),
        pltpu.VMEM((NCH, C), jnp.int32),
        pltpu.VMEM((C,), jnp.int32),
        pltpu.VMEM((C,), jnp.int32),
        pltpu.VMEM((C, D), jnp.float32),
        pltpu.VMEM_SHARED((ACC_ROWS, D), jnp.float32),
        pltpu.SemaphoreType.DMA,
    ]
    return pl.kernel(
        functools.partial(_sc_body, with_cnt),
        out_type=tuple(out_type),
        mesh=mesh,
        scratch_types=scratch,
        compiler_params=pltpu.CompilerParams(needs_layout_passes=False),
    )


_make_sc = functools.lru_cache(maxsize=None)(_make_sc)


def _tc_body(h_ref, a_ref, cnt_ref, wrel_ref, wroot_ref, bias_ref, g_ref,
             be_ref, o_ref):
    h = h_ref[...]
    out = jnp.dot(h, wroot_ref[...], preferred_element_type=jnp.float32)
    out = out + bias_ref[...]
    for r in range(R):
        a = a_ref[0, r] + a_ref[1, r]
        c = (cnt_ref[0, r] + cnt_ref[1, r])[:, 0:1]
        inv = 1.0 / jnp.maximum(c, 1.0)
        out = out + jnp.dot(a * inv, wrel_ref[r],
                            preferred_element_type=jnp.float32)
    mu = jnp.mean(out, axis=1, keepdims=True)
    xc = out - mu
    var = jnp.mean(xc * xc, axis=1, keepdims=True)
    y = xc * lax.rsqrt(var + 1e-5)
    y = y * g_ref[...] + be_ref[...]
    o_ref[...] = jnp.maximum(y, 0.0)


def _tc_layer(h, a_p, cnt_p, wrel, wroot, bias, g, be):
    nb = NPAD // 128
    return pl.pallas_call(
        _tc_body,
        grid=(nb,),
        in_specs=[
            pl.BlockSpec((128, D), lambda b: (b, 0)),
            pl.BlockSpec((NC, R, 128, D), lambda b: (0, 0, b, 0)),
            pl.BlockSpec((NC, R, 128, D), lambda b: (0, 0, b, 0)),
            pl.BlockSpec((R, D, D), lambda b: (0, 0, 0)),
            pl.BlockSpec((D, D), lambda b: (0, 0)),
            pl.BlockSpec((1, D), lambda b: (0, 0)),
            pl.BlockSpec((1, D), lambda b: (0, 0)),
            pl.BlockSpec((1, D), lambda b: (0, 0)),
        ],
        out_specs=pl.BlockSpec((128, D), lambda b: (b, 0)),
        out_shape=jax.ShapeDtypeStruct((NPAD, D), jnp.float32),
    )(h, a_p, cnt_p, wrel, wroot, bias, g, be)


def kernel(x, edge_index, edge_attr, Wrel0, Wroot0, b0, g0, be0,
           Wrel1, Wroot1, b1, g1, be1):
    src = edge_index[0]
    dst = edge_index[1]
    et = edge_attr[:, 0]
    epk = src | (dst << 14) | (et << 28)

    h0 = jnp.pad(x, ((0, NPAD - N), (0, 0)))
    z656 = jnp.zeros((656, D), jnp.float32)
    o128 = jnp.ones((C, D), jnp.float32)

    a0, cnt_p = _make_sc(True)(h0, epk, z656, o128)

    h1 = _tc_layer(h0, a0, cnt_p, Wrel0, Wroot0, b0.reshape(1, D),
                   g0.reshape(1, D), be0.reshape(1, D))
    (a1,) = _make_sc(False)(h1, epk, z656, o128)
    h2 = _tc_layer(h1, a1, cnt_p, Wrel1, Wroot1, b1.reshape(1, D),
                   g1.reshape(1, D), be1.reshape(1, D))
    return jnp.stack([h1[:N], h2[:N]])


# speedup vs baseline: 8.6979x; 1.4896x over previous
import functools

import jax
import jax.numpy as jnp
from jax import lax
from jax.experimental import pallas as pl
from jax.experimental.pallas import tpu as pltpu
from jax.experimental.pallas import tpu_sc as plsc

N = 10000
E = 320000
D = 128
R = 4

NPAD = 10240
NPASS = 4
RANGE = NPAD // NPASS
SLOTS = R * RANGE
TRASH = SLOTS
ACC_ROWS = 10496
NC = 2
NS = 16
NT = NC * NS
EPT = E // NT
NV = EPT // 16
C = 64
SEL_ROWS = 80
MASK14 = (1 << 14) - 1


def _sc_common(scr):
    (sel, src_c0, src_c1, slot_c0, slot_c1, rows0, rows1, nch_buf, acc,
     sem0, sem1) = scr
    src_c = (src_c0, src_c1)
    slot_c = (slot_c0, slot_c1)
    rows = (rows0, rows1)
    sem = (sem0, sem1)

    def unpack(j, b, slot_only=False):
        row = jnp.right_shift(j, 1)
        cb = jnp.bitwise_and(j, 1) * C
        for k in range(C // 16):
            pp = sel[row, pl.ds(cb + k * 16, 16)]
            if not slot_only:
                src_c[b][pl.ds(k * 16, 16)] = jnp.bitwise_and(pp, MASK14)
            slot_c[b][pl.ds(k * 16, 16)] = jnp.right_shift(pp, 14)

    def run_chunks(nch, h):
        @pl.when(nch > 0)
        def _():
            unpack(0, 0)
            pltpu.async_copy(h.at[src_c[0]], rows[0], sem[0])

        def outer(t, carry):
            for b in range(2):
                j = 2 * t + b

                @pl.when(j < nch)
                def _():
                    @pl.when(j + 1 < nch)
                    def _():
                        unpack(j + 1, 1 - b)
                        pltpu.async_copy(h.at[src_c[1 - b]], rows[1 - b],
                                         sem[1 - b])
                    pltpu.make_async_copy(h.at[src_c[b]], rows[b],
                                          sem[b]).wait()
                    pltpu.sync_copy(rows[b], acc.at[slot_c[b]], add=True)
            return carry

        lax.fori_loop(0, (nch + 1) // 2, outer, jnp.int32(0))

    def run_cnt_chunks(nch):
        def outer(t, carry):
            for b in range(2):
                j = 2 * t + b

                @pl.when(j < nch)
                def _():
                    @pl.when(j >= 2)
                    def _():
                        pltpu.make_async_copy(rows[0], acc.at[slot_c[b]],
                                              sem[b]).wait()
                    unpack(j, b, slot_only=True)
                    pltpu.async_copy(rows[0], acc.at[slot_c[b]], sem[b],
                                     add=True)
            return carry

        lax.fori_loop(0, (nch + 1) // 2, outer, jnp.int32(0))

        @pl.when(nch >= 2)
        def _():
            pltpu.make_async_copy(rows[0], acc.at[slot_c[0]], sem[0]).wait()
            pltpu.make_async_copy(rows[0], acc.at[slot_c[1]], sem[1]).wait()

        @pl.when(nch == 1)
        def _():
            pltpu.make_async_copy(rows[0], acc.at[slot_c[0]], sem[0]).wait()

    return sel, rows, nch_buf, acc, run_chunks, run_cnt_chunks


def _writeout(acc, dst_ref, ci, si, lo):
    r_idx = si // 4
    noff = (si % 4) * 640
    pltpu.sync_copy(acc.at[pl.ds(si * 640, 640)],
                    dst_ref.at[ci, r_idx, pl.ds(lo + noff, 640)])


def _sc0_body(h, epk, z656, o64, a_out, cnt_out, sel_out, nch_out, e_all,
              *scr):
    sel, rows, nch_buf, acc, run_chunks, run_cnt_chunks = _sc_common(scr)
    ci = lax.axis_index("c")
    si = lax.axis_index("s")
    tid = ci * NS + si

    pltpu.sync_copy(epk.at[pl.ds(tid * EPT, EPT)], e_all)

    it16 = lax.iota(jnp.int32, 16)
    nch_v = jnp.zeros((16,), jnp.int32)
    nchs = []
    for rho in range(NPASS):
        pltpu.sync_copy(z656, acc.at[pl.ds(si * 656, 656)])
        plsc.subcore_barrier()

        lo = rho * RANGE

        def scan_body(i, cnt):
            p = e_all[pl.ds(i * 16, 16)]
            src_v = jnp.bitwise_and(p, MASK14)
            dst_v = jnp.bitwise_and(jnp.right_shift(p, 14), MASK14)
            et_v = jnp.right_shift(p, 28)
            m = (dst_v >= lo) & (dst_v < lo + RANGE)
            slot_v = et_v * RANGE + (dst_v - lo)
            entry = jnp.bitwise_or(src_v, jnp.left_shift(slot_v, 14))
            mi = m.astype(jnp.int32)
            pf = plsc.cumsum(mi)
            q = cnt + pf - 1
            ri = jnp.right_shift(q, 7)
            co = jnp.bitwise_and(q, 127)
            plsc.store_scatter(sel, [ri, co], entry, mask=m)
            return cnt + jnp.sum(mi)

        cnt = lax.fori_loop(0, NV, scan_body, jnp.int32(0))

        trash_v = jnp.full((16,), TRASH << 14, jnp.int32)
        for k in range(C // 16):
            q = cnt + k * 16 + it16
            ri = jnp.right_shift(q, 7)
            co = jnp.bitwise_and(q, 127)
            plsc.store_scatter(sel, [ri, co], trash_v)

        nch = (cnt + (C - 1)) // C
        nchs.append(nch)
        nch_v = jnp.where(it16 == rho, nch, nch_v)

        pltpu.sync_copy(sel, sel_out.at[tid, rho])

        run_chunks(nch, h)
        plsc.subcore_barrier()
        _writeout(acc, a_out, ci, si, lo)
        plsc.subcore_barrier()

    nch_buf[pl.ds(0, 16)] = nch_v
    pltpu.sync_copy(nch_buf, nch_out.at[tid])

    pltpu.sync_copy(o64, rows[0])
    for rho in range(NPASS):
        pltpu.sync_copy(z656, acc.at[pl.ds(si * 656, 656)])
        plsc.subcore_barrier()
        pltpu.sync_copy(sel_out.at[tid, rho], sel)
        run_cnt_chunks(nchs[rho])
        plsc.subcore_barrier()
        _writeout(acc, cnt_out, ci, si, rho * RANGE)
        plsc.subcore_barrier()


def _sc1_body(h, sel_in, nch_in, z656, a_out, *scr):
    sel, rows, nch_buf, acc, run_chunks, _ = _sc_common(scr)
    ci = lax.axis_index("c")
    si = lax.axis_index("s")
    tid = ci * NS + si

    pltpu.sync_copy(nch_in.at[tid], nch_buf)
    it16 = lax.iota(jnp.int32, 16)
    nch_v = nch_buf[pl.ds(0, 16)]

    for rho in range(NPASS):
        nch = jnp.sum(jnp.where(it16 == rho, nch_v, 0))
        pltpu.sync_copy(z656, acc.at[pl.ds(si * 656, 656)])
        plsc.subcore_barrier()
        pltpu.sync_copy(sel_in.at[tid, rho], sel)
        run_chunks(nch, h)
        plsc.subcore_barrier()
        _writeout(acc, a_out, ci, si, rho * RANGE)
        plsc.subcore_barrier()


def _common_scratch():
    return [
        pltpu.VMEM((SEL_ROWS, 128), jnp.int32),
        pltpu.VMEM((C,), jnp.int32),
        pltpu.VMEM((C,), jnp.int32),
        pltpu.VMEM((C,), jnp.int32),
        pltpu.VMEM((C,), jnp.int32),
        pltpu.VMEM((C, D), jnp.float32),
        pltpu.VMEM((C, D), jnp.float32),
        pltpu.VMEM((16,), jnp.int32),
        pltpu.VMEM_SHARED((ACC_ROWS, D), jnp.float32),
        pltpu.SemaphoreType.DMA,
        pltpu.SemaphoreType.DMA,
    ]


def _mesh():
    return plsc.VectorSubcoreMesh(core_axis_name="c", subcore_axis_name="s",
                                  num_cores=NC, num_subcores=NS)


@functools.lru_cache(maxsize=None)
def _make_sc0():
    return pl.kernel(
        _sc0_body,
        out_type=(
            jax.ShapeDtypeStruct((NC, R, NPAD, D), jnp.float32),
            jax.ShapeDtypeStruct((NC, R, NPAD, D), jnp.float32),
            jax.ShapeDtypeStruct((NT, NPASS, SEL_ROWS, 128), jnp.int32),
            jax.ShapeDtypeStruct((NT, 16), jnp.int32),
        ),
        mesh=_mesh(),
        scratch_types=[pltpu.VMEM((EPT,), jnp.int32)] + _common_scratch(),
        compiler_params=pltpu.CompilerParams(needs_layout_passes=False),
    )


@functools.lru_cache(maxsize=None)
def _make_sc1():
    return pl.kernel(
        _sc1_body,
        out_type=(jax.ShapeDtypeStruct((NC, R, NPAD, D), jnp.float32),),
        mesh=_mesh(),
        scratch_types=_common_scratch(),
        compiler_params=pltpu.CompilerParams(needs_layout_passes=False),
    )


def _tc_body(h_ref, a_ref, cnt_ref, wrel_ref, wroot_ref, bias_ref, g_ref,
             be_ref, o_ref):
    h = h_ref[...]
    out = jnp.dot(h, wroot_ref[...], preferred_element_type=jnp.float32)
    out = out + bias_ref[...]
    for r in range(R):
        a = a_ref[0, r] + a_ref[1, r]
        c = (cnt_ref[0, r] + cnt_ref[1, r])[:, 0:1]
        inv = 1.0 / jnp.maximum(c, 1.0)
        out = out + jnp.dot(a * inv, wrel_ref[r],
                            preferred_element_type=jnp.float32)
    mu = jnp.mean(out, axis=1, keepdims=True)
    xc = out - mu
    var = jnp.mean(xc * xc, axis=1, keepdims=True)
    y = xc * lax.rsqrt(var + 1e-5)
    y = y * g_ref[...] + be_ref[...]
    o_ref[...] = jnp.maximum(y, 0.0)


def _tc_layer(h, a_p, cnt_p, wrel, wroot, bias, g, be):
    nb = NPAD // 128
    return pl.pallas_call(
        _tc_body,
        grid=(nb,),
        in_specs=[
            pl.BlockSpec((128, D), lambda b: (b, 0)),
            pl.BlockSpec((NC, R, 128, D), lambda b: (0, 0, b, 0)),
            pl.BlockSpec((NC, R, 128, D), lambda b: (0, 0, b, 0)),
            pl.BlockSpec((R, D, D), lambda b: (0, 0, 0)),
            pl.BlockSpec((D, D), lambda b: (0, 0)),
            pl.BlockSpec((1, D), lambda b: (0, 0)),
            pl.BlockSpec((1, D), lambda b: (0, 0)),
            pl.BlockSpec((1, D), lambda b: (0, 0)),
        ],
        out_specs=pl.BlockSpec((128, D), lambda b: (b, 0)),
        out_shape=jax.ShapeDtypeStruct((NPAD, D), jnp.float32),
    )(h, a_p, cnt_p, wrel, wroot, bias, g, be)


def kernel(x, edge_index, edge_attr, Wrel0, Wroot0, b0, g0, be0,
           Wrel1, Wroot1, b1, g1, be1):
    src = edge_index[0]
    dst = edge_index[1]
    et = edge_attr[:, 0]
    epk = src | (dst << 14) | (et << 28)

    h0 = jnp.pad(x, ((0, NPAD - N), (0, 0)))
    z656 = jnp.zeros((656, D), jnp.float32)
    o64 = jnp.ones((C, D), jnp.float32)

    a0, cnt_p, sel_s, nch_s = _make_sc0()(h0, epk, z656, o64)

    h1 = _tc_layer(h0, a0, cnt_p, Wrel0, Wroot0, b0.reshape(1, D),
                   g0.reshape(1, D), be0.reshape(1, D))
    (a1,) = _make_sc1()(h1, sel_s, nch_s, z656)
    h2 = _tc_layer(h1, a1, cnt_p, Wrel1, Wroot1, b1.reshape(1, D),
                   g1.reshape(1, D), be1.reshape(1, D))
    return jnp.stack([h1[:N], h2[:N]])
